# baseline (speedup 1.0000x reference)
import functools

import jax
import jax.numpy as jnp
from jax import lax
from jax.experimental import pallas as pl
from jax.experimental.pallas import tpu as pltpu
from jax.experimental.pallas import tpu_sc as plsc

NC = 2
NS = 16
NW = NC * NS
L = 16

EPS = 1e-12
CHUNK = 48
NBUF = 2


def _embed_ln(ids, word, pos, ttype, gamma, beta, *, n_tok, hid, seq):
    tpw = n_tok // NW
    sizes = [CHUNK] * (tpw // CHUNK)
    if tpw % CHUNK:
        sizes.append(tpw % CHUNK)
    starts = [sum(sizes[:i]) for i in range(len(sizes))]
    n_chunks = len(sizes)
    mesh = plsc.VectorSubcoreMesh(core_axis_name="c", subcore_axis_name="s")

    @functools.partial(
        pl.kernel,
        out_type=jax.ShapeDtypeStruct((n_tok, hid), jnp.float32),
        mesh=mesh,
        scratch_types=[
            pltpu.VMEM((tpw,), jnp.int32),
            [pltpu.VMEM((CHUNK, hid), jnp.float32)] * NBUF,
            pltpu.VMEM((CHUNK, hid), jnp.float32),
            pltpu.VMEM((hid,), jnp.float32),
            pltpu.VMEM((hid,), jnp.float32),
            pltpu.VMEM((hid,), jnp.float32),
            [pltpu.SemaphoreType.DMA] * NBUF,
            pltpu.SemaphoreType.DMA,
            [pltpu.SemaphoreType.DMA] * NBUF,
        ],
        compiler_params=pltpu.CompilerParams(use_tc_tiling_on_sc=True,
                                             needs_layout_passes=False,
                                             disable_semaphore_checks=True,
                                             disable_bounds_checks=True,
                                             skip_device_barrier=True),
    )
    def body(ids_hbm, w_hbm, p_hbm, t_hbm, g_hbm, b_hbm, out_hbm,
             idx_all, wbufs, pbuf, tb, gb, bb, semg, semp, semo):
        wid = lax.axis_index("s") * NC + lax.axis_index("c")
        base = wid * tpw

        pltpu.sync_copy(ids_hbm.at[pl.ds(base, tpw)], idx_all)
        pltpu.sync_copy(t_hbm.at[0], tb)
        pltpu.sync_copy(g_hbm, gb)
        pltpu.sync_copy(b_hbm, bb)

        inv_h = jnp.float32(1.0 / hid)
        zero = jnp.zeros((L,), jnp.float32)
        lane = lax.iota(jnp.int32, 16)
        rows0 = [lane + g * L for g in range(CHUNK // L)]

        def issue_gather(c, b):
            return pltpu.async_copy(
                w_hbm.at[idx_all.at[pl.ds(starts[c], sizes[c])]],
                wbufs[b].at[pl.ds(0, sizes[c])], semg[b])

        def issue_pos(c):
            s0 = lax.rem(base + starts[c], seq)
            return pltpu.async_copy(p_hbm.at[pl.ds(s0, sizes[c])],
                                    pbuf.at[pl.ds(0, sizes[c])], semp)

        inflight = [None] * NBUF
        outflight = [None] * NBUF

        inflight[0] = issue_gather(0, 0)
        pflight = issue_pos(0)

        for c in range(n_chunks):
            b = c % NBUF
            nb = (c + 1) % NBUF
            ngrp = sizes[c] // L
            if c + 1 < n_chunks:
                if outflight[nb] is not None:
                    outflight[nb].wait()
                    outflight[nb] = None
                inflight[nb] = issue_gather(c + 1, nb)
            inflight[b].wait()
            pflight.wait()
            wbuf = wbufs[b]

            @plsc.parallel_loop(0, hid, unroll=4,
                                carry=tuple([zero] * (2 * ngrp)))
            def _p1(j, acc, wbuf=wbuf, ngrp=ngrp):
                jf = jnp.full((L,), j, jnp.int32) ^ lane
                tv = plsc.load_gather(tb, [jf])
                out = []
                for g in range(ngrp):
                    wv = plsc.load_gather(wbuf, [rows0[g], jf])
                    pv = plsc.load_gather(pbuf, [rows0[g], jf])
                    x = wv + pv + tv
                    plsc.store_scatter(wbuf, [rows0[g], jf], x)
                    out.append(acc[2 * g] + x)
                    out.append(acc[2 * g + 1] + x * x)
                return tuple(out)

            if c + 1 < n_chunks:
                pflight = issue_pos(c + 1)

            stats = []
            for g in range(ngrp):
                mu = _p1[2 * g] * inv_h
                var = _p1[2 * g + 1] * inv_h - mu * mu
                v = var + jnp.float32(EPS)
                bits = plsc.bitcast(v, jnp.int32)
                y = plsc.bitcast(jnp.int32(0x5F3759DF) - (bits >> 1),
                                 jnp.float32)
                for _ in range(3):
                    y = y * (jnp.float32(1.5) - jnp.float32(0.5) * v * y * y)
                stats.append((mu, y))

            @plsc.parallel_loop(0, hid, unroll=4)
            def _p2(j, wbuf=wbuf, stats=stats, ngrp=ngrp):
                jf = jnp.full((L,), j, jnp.int32) ^ lane
                gv = plsc.load_gather(gb, [jf])
                bv = plsc.load_gather(bb, [jf])
                for g in range(ngrp):
                    x = plsc.load_gather(wbuf, [rows0[g], jf])
                    mu, y = stats[g]
                    o = (x - mu) * y * gv + bv
                    plsc.store_scatter(wbuf, [rows0[g], jf], o)

            outflight[b] = pltpu.async_copy(
                wbuf.at[pl.ds(0, sizes[c])],
                out_hbm.at[pl.ds(base + starts[c], sizes[c])], semo[b])

        for b in range(NBUF):
            if outflight[b] is not None:
                outflight[b].wait()

    return body(ids, word, pos, ttype, gamma, beta)


def kernel(input_ids, turn_ids, position_ids, role_ids, word_embeddings,
           position_embeddings, token_type_embeddings, ln_gamma, ln_beta):
    b, s = input_ids.shape
    hid = word_embeddings.shape[1]
    ids = input_ids.reshape(-1).astype(jnp.int32)
    out = _embed_ln(ids, word_embeddings, position_embeddings,
                    token_type_embeddings, ln_gamma, ln_beta,
                    n_tok=b * s, hid=hid, seq=s)
    return out.reshape(b, s, hid)


# speedup vs baseline: 1.0064x; 1.0064x over previous
import functools

import jax
import jax.numpy as jnp
from jax import lax
from jax.experimental import pallas as pl
from jax.experimental.pallas import tpu as pltpu
from jax.experimental.pallas import tpu_sc as plsc

NC = 2
NS = 16
NW = NC * NS
L = 16

EPS = 1e-12
CHUNK = 48
NBUF = 2


def _embed_ln(ids, word, pos, ttype, gamma, beta, *, n_tok, hid, seq):
    tpw = n_tok // NW
    sizes = [CHUNK] * (tpw // CHUNK)
    if tpw % CHUNK:
        sizes.append(tpw % CHUNK)
    starts = [sum(sizes[:i]) for i in range(len(sizes))]
    n_chunks = len(sizes)
    mesh = plsc.VectorSubcoreMesh(core_axis_name="c", subcore_axis_name="s")

    @functools.partial(
        pl.kernel,
        out_type=jax.ShapeDtypeStruct((n_tok, hid), jnp.float32),
        mesh=mesh,
        scratch_types=[
            pltpu.VMEM((tpw,), jnp.int32),
            [pltpu.VMEM((CHUNK, hid), jnp.float32)] * NBUF,
            pltpu.VMEM((CHUNK, hid), jnp.float32),
            pltpu.VMEM((hid,), jnp.float32),
            pltpu.VMEM((hid,), jnp.float32),
            pltpu.VMEM((hid,), jnp.float32),
            [pltpu.SemaphoreType.DMA] * NBUF,
            pltpu.SemaphoreType.DMA,
            [pltpu.SemaphoreType.DMA] * NBUF,
        ],
        compiler_params=pltpu.CompilerParams(use_tc_tiling_on_sc=True,
                                             needs_layout_passes=False,
                                             disable_semaphore_checks=True,
                                             disable_bounds_checks=True),
    )
    def body(ids_hbm, w_hbm, p_hbm, t_hbm, g_hbm, b_hbm, out_hbm,
             idx_all, wbufs, pbuf, tb, gb, bb, semg, semp, semo):
        wid = lax.axis_index("s") * NC + lax.axis_index("c")
        base = wid * tpw

        pltpu.sync_copy(ids_hbm.at[pl.ds(base, tpw)], idx_all)
        pltpu.sync_copy(t_hbm.at[0], tb)
        pltpu.sync_copy(g_hbm, gb)
        pltpu.sync_copy(b_hbm, bb)

        inv_h = jnp.float32(1.0 / hid)
        zero = jnp.zeros((L,), jnp.float32)
        lane = lax.iota(jnp.int32, 16)
        rows0 = [lane + g * L for g in range(CHUNK // L)]

        def issue_gather(c, b):
            return pltpu.async_copy(
                w_hbm.at[idx_all.at[pl.ds(starts[c], sizes[c])]],
                wbufs[b].at[pl.ds(0, sizes[c])], semg[b])

        def issue_pos(c):
            s0 = lax.rem(base + starts[c], seq)
            return pltpu.async_copy(p_hbm.at[pl.ds(s0, sizes[c])],
                                    pbuf.at[pl.ds(0, sizes[c])], semp)

        inflight = [None] * NBUF
        outflight = [None] * NBUF

        inflight[0] = issue_gather(0, 0)
        pflight = issue_pos(0)

        for c in range(n_chunks):
            b = c % NBUF
            nb = (c + 1) % NBUF
            ngrp = sizes[c] // L
            if c + 1 < n_chunks:
                if outflight[nb] is not None:
                    outflight[nb].wait()
                    outflight[nb] = None
                inflight[nb] = issue_gather(c + 1, nb)
            inflight[b].wait()
            pflight.wait()
            wbuf = wbufs[b]

            @plsc.parallel_loop(0, hid, unroll=4,
                                carry=tuple([zero] * (2 * ngrp)))
            def _p1(j, acc, wbuf=wbuf, ngrp=ngrp):
                jf = jnp.full((L,), j, jnp.int32) ^ lane
                tv = plsc.load_gather(tb, [jf])
                out = []
                for g in range(ngrp):
                    wv = plsc.load_gather(wbuf, [rows0[g], jf])
                    pv = plsc.load_gather(pbuf, [rows0[g], jf])
                    x = wv + pv + tv
                    plsc.store_scatter(wbuf, [rows0[g], jf], x)
                    out.append(acc[2 * g] + x)
                    out.append(acc[2 * g + 1] + x * x)
                return tuple(out)

            if c + 1 < n_chunks:
                pflight = issue_pos(c + 1)

            stats = []
            for g in range(ngrp):
                mu = _p1[2 * g] * inv_h
                var = _p1[2 * g + 1] * inv_h - mu * mu
                v = var + jnp.float32(EPS)
                bits = plsc.bitcast(v, jnp.int32)
                y = plsc.bitcast(jnp.int32(0x5F3759DF) - (bits >> 1),
                                 jnp.float32)
                for _ in range(4):
                    y = y * (jnp.float32(1.5) - jnp.float32(0.5) * v * y * y)
                stats.append((mu, y))

            @plsc.parallel_loop(0, hid, unroll=4)
            def _p2(j, wbuf=wbuf, stats=stats, ngrp=ngrp):
                jf = jnp.full((L,), j, jnp.int32) ^ lane
                gv = plsc.load_gather(gb, [jf])
                bv = plsc.load_gather(bb, [jf])
                for g in range(ngrp):
                    x = plsc.load_gather(wbuf, [rows0[g], jf])
                    mu, y = stats[g]
                    o = (x - mu) * y * gv + bv
                    plsc.store_scatter(wbuf, [rows0[g], jf], o)

            outflight[b] = pltpu.async_copy(
                wbuf.at[pl.ds(0, sizes[c])],
                out_hbm.at[pl.ds(base + starts[c], sizes[c])], semo[b])

        for b in range(NBUF):
            if outflight[b] is not None:
                outflight[b].wait()

    return body(ids, word, pos, ttype, gamma, beta)


def kernel(input_ids, turn_ids, position_ids, role_ids, word_embeddings,
           position_embeddings, token_type_embeddings, ln_gamma, ln_beta):
    b, s = input_ids.shape
    hid = word_embeddings.shape[1]
    ids = input_ids.reshape(-1).astype(jnp.int32)
    out = _embed_ln(ids, word_embeddings, position_embeddings,
                    token_type_embeddings, ln_gamma, ln_beta,
                    n_tok=b * s, hid=hid, seq=s)
    return out.reshape(b, s, hid)
